# baseline (device time: 108195 ns/iter reference)
import jax
import jax.numpy as jnp
from jax import lax
from jax.experimental import pallas as pl
from jax.experimental.pallas import tpu as pltpu

N_DEV = 8
SQ = 1024
SKV = 1024
H_PER = 8
DH = 128
DM = 1024
BLK = 64
SCALE = 0.08838834764831843


def kernel(x, Wq, K_ext, V_ext, Wo):
    my = lax.axis_index("i")

    xb = x[0].astype(jnp.bfloat16)
    wq_i = lax.dynamic_slice(
        Wq, (0, my * (H_PER * DH)), (DM, H_PER * DH)
    ).astype(jnp.bfloat16)
    k_t = K_ext[0].transpose(1, 2, 0).astype(jnp.bfloat16)
    v_t = V_ext[0].transpose(1, 0, 2).astype(jnp.bfloat16)
    wo_i = lax.dynamic_slice(
        Wo, (my * (H_PER * DH), 0), (H_PER * DH, DM)
    ).astype(jnp.bfloat16)

    def body(x_ref, wq_ref, k_ref, v_ref, wo_ref, out_ref,
             send_buf, recv_buf, send_sems, recv_sems):
        my_pos = lax.axis_index("i")

        partners = [
            my_pos ^ 1,
            (my_pos & 4) | ((my_pos & 3) ^ 3),
            my_pos ^ 4,
        ]

        barrier_sem = pltpu.get_barrier_semaphore()
        for p in partners:
            pl.semaphore_signal(
                barrier_sem, inc=1,
                device_id=(p,), device_id_type=pl.DeviceIdType.MESH,
            )
        pl.semaphore_wait(barrier_sem, 3)

        q = jnp.dot(
            x_ref[:, :], wq_ref[:, :], preferred_element_type=jnp.float32
        ).astype(jnp.bfloat16)

        qb = lax.broadcasted_iota(jnp.int32, (SQ, SKV), 0) // BLK
        kb = lax.broadcasted_iota(jnp.int32, (SQ, SKV), 1) // BLK
        mask = kb <= qb

        ctx_heads = []
        for h in range(H_PER):
            q_h = q[:, h * DH:(h + 1) * DH]
            s = jnp.dot(
                q_h, k_ref[h], preferred_element_type=jnp.float32
            ) * SCALE
            s = jnp.where(mask, s, -1e9)
            m = jnp.max(s, axis=1, keepdims=True)
            e = jnp.exp(s - m)
            w = (e / jnp.sum(e, axis=1, keepdims=True)).astype(jnp.bfloat16)
            ctx_heads.append(jnp.dot(
                w, v_ref[h], preferred_element_type=jnp.float32
            ).astype(jnp.bfloat16))
        ctx = jnp.concatenate(ctx_heads, axis=1)

        acc = jnp.dot(ctx, wo_ref[:, :], preferred_element_type=jnp.float32)

        for r in range(3):
            send_buf[:, :] = acc.astype(jnp.bfloat16)
            rdma = pltpu.make_async_remote_copy(
                src_ref=send_buf,
                dst_ref=recv_buf.at[r],
                send_sem=send_sems.at[r],
                recv_sem=recv_sems.at[r],
                device_id=(partners[r],),
                device_id_type=pl.DeviceIdType.MESH,
            )
            rdma.start()
            rdma.wait()
            acc = acc + recv_buf[r, :, :].astype(jnp.float32)

        out_ref[0, :, :] = acc

    out = pl.pallas_call(
        body,
        out_shape=jax.ShapeDtypeStruct((1, SQ, DM), jnp.float32),
        in_specs=[pl.BlockSpec(memory_space=pltpu.VMEM)] * 5,
        out_specs=pl.BlockSpec(memory_space=pltpu.VMEM),
        scratch_shapes=[
            pltpu.VMEM((SQ, DM), jnp.bfloat16),
            pltpu.VMEM((3, SQ, DM), jnp.bfloat16),
            pltpu.SemaphoreType.DMA((3,)),
            pltpu.SemaphoreType.DMA((3,)),
        ],
        compiler_params=pltpu.CompilerParams(collective_id=0),
    )(xb, wq_i, k_t, v_t, wo_i)
    return out


# device time: 60411 ns/iter; 1.7910x vs baseline; 1.7910x over previous
import jax
import jax.numpy as jnp
from jax import lax
from jax.experimental import pallas as pl
from jax.experimental.pallas import tpu as pltpu

N_DEV = 8
SQ = 1024
SKV = 1024
H_PER = 8
DH = 128
DM = 1024
BLK = 64
SCALE = 0.08838834764831843


def kernel(x, Wq, K_ext, V_ext, Wo):
    my = lax.axis_index("i")

    xb = x[0].astype(jnp.bfloat16)
    wq_i = lax.dynamic_slice(
        Wq, (0, my * (H_PER * DH)), (DM, H_PER * DH)
    ).astype(jnp.bfloat16)
    k_t = K_ext[0].transpose(1, 2, 0).astype(jnp.bfloat16)
    v_t = V_ext[0].transpose(1, 0, 2).astype(jnp.bfloat16)
    wo_i = lax.dynamic_slice(
        Wo, (my * (H_PER * DH), 0), (H_PER * DH, DM)
    ).astype(jnp.bfloat16)

    PARTS = [(0, 384), (384, 384), (768, 256)]
    RS_OFF = [
        [sum(PARTS[u][1] >> (r + 1) for u in range(t)) for t in range(3)]
        for r in range(3)
    ]
    RS_TOT = [sum(PARTS[t][1] >> (r + 1) for t in range(3)) for r in range(3)]

    def body(x_ref, wq_ref, k_ref, v_ref, wo_ref, out_ref,
             work, rs_recv0, rs_recv1, rs_recv2,
             rs_send_sems, rs_recv_sems, ag_send_sems, ag_recv_sems):
        my_pos = lax.axis_index("i")

        partners = [
            my_pos ^ 1,
            (my_pos & 4) | ((my_pos & 3) ^ 3),
            my_pos ^ 4,
        ]
        bits = [
            (my_pos ^ (my_pos >> 1)) & 1,
            (my_pos >> 1) & 1,
            (my_pos >> 2) & 1,
        ]
        rs_recv = [rs_recv0, rs_recv1, rs_recv2]

        barrier_sem = pltpu.get_barrier_semaphore()
        for p in partners:
            pl.semaphore_signal(
                barrier_sem, inc=1,
                device_id=(p,), device_id_type=pl.DeviceIdType.MESH,
            )
        pl.semaphore_wait(barrier_sem, 3)

        q = jnp.dot(
            x_ref[:, :], wq_ref[:, :], preferred_element_type=jnp.float32
        ).astype(jnp.bfloat16)

        qb = lax.broadcasted_iota(jnp.int32, (SQ, SKV), 0) // BLK
        kb = lax.broadcasted_iota(jnp.int32, (SQ, SKV), 1) // BLK
        mask = kb <= qb

        ctx_heads = []
        for h in range(H_PER):
            q_h = q[:, h * DH:(h + 1) * DH]
            s = jnp.dot(
                q_h, k_ref[h], preferred_element_type=jnp.float32
            ) * SCALE
            s = jnp.where(mask, s, -1e9)
            m = jnp.max(s, axis=1, keepdims=True)
            e = jnp.exp(s - m)
            w = (e / jnp.sum(e, axis=1, keepdims=True)).astype(jnp.bfloat16)
            ctx_heads.append(jnp.dot(
                w, v_ref[h], preferred_element_type=jnp.float32
            ).astype(jnp.bfloat16))
        ctx = jnp.concatenate(ctx_heads, axis=1)

        acc = jnp.dot(ctx, wo_ref[:, :], preferred_element_type=jnp.float32)
        work[:, :] = acc.astype(jnp.bfloat16)

        state = [(jnp.int32(p0), n) for (p0, n) in PARTS]
        for r in range(3):
            inflight = []
            for t in range(3):
                d = (t + r) % 3
                start, sz = state[t]
                half = sz // 2
                b = bits[d]
                keep_start = start + b * half
                send_start = start + (1 - b) * half
                rdma = pltpu.make_async_remote_copy(
                    src_ref=work.at[pl.ds(send_start, half), :],
                    dst_ref=rs_recv[r].at[pl.ds(RS_OFF[r][t], half), :],
                    send_sem=rs_send_sems.at[t, r],
                    recv_sem=rs_recv_sems.at[t, r],
                    device_id=(partners[d],),
                    device_id_type=pl.DeviceIdType.MESH,
                )
                rdma.start()
                inflight.append((rdma, t, keep_start, half))
                state[t] = (keep_start, half)
            for rdma, t, keep_start, half in inflight:
                rdma.wait()
                work[pl.ds(keep_start, half), :] = (
                    work[pl.ds(keep_start, half), :]
                    + rs_recv[r][pl.ds(RS_OFF[r][t], half), :]
                )

        for r in range(3):
            inflight = []
            for t in range(3):
                d = (t + 2 - r) % 3
                start, sz = state[t]
                rdma = pltpu.make_async_remote_copy(
                    src_ref=work.at[pl.ds(start, sz), :],
                    dst_ref=work.at[pl.ds(start, sz), :],
                    send_sem=ag_send_sems.at[t, r],
                    recv_sem=ag_recv_sems.at[t, r],
                    device_id=(partners[d],),
                    device_id_type=pl.DeviceIdType.MESH,
                )
                rdma.start()
                inflight.append(rdma)
                state[t] = (start - bits[d] * sz, 2 * sz)
            for rdma in inflight:
                rdma.wait()

        out_ref[0, :, :] = work[:, :].astype(jnp.float32)

    out = pl.pallas_call(
        body,
        out_shape=jax.ShapeDtypeStruct((1, SQ, DM), jnp.float32),
        in_specs=[pl.BlockSpec(memory_space=pltpu.VMEM)] * 5,
        out_specs=pl.BlockSpec(memory_space=pltpu.VMEM),
        scratch_shapes=[
            pltpu.VMEM((SQ, DM), jnp.bfloat16),
            pltpu.VMEM((RS_TOT[0], DM), jnp.bfloat16),
            pltpu.VMEM((RS_TOT[1], DM), jnp.bfloat16),
            pltpu.VMEM((RS_TOT[2], DM), jnp.bfloat16),
            pltpu.SemaphoreType.DMA((3, 3)),
            pltpu.SemaphoreType.DMA((3, 3)),
            pltpu.SemaphoreType.DMA((3, 3)),
            pltpu.SemaphoreType.DMA((3, 3)),
        ],
        compiler_params=pltpu.CompilerParams(collective_id=0),
    )(xb, wq_i, k_t, v_t, wo_i)
    return out


# device time: 59583 ns/iter; 1.8159x vs baseline; 1.0139x over previous
import jax
import jax.numpy as jnp
from jax import lax
from jax.experimental import pallas as pl
from jax.experimental.pallas import tpu as pltpu

N_DEV = 8
SQ = 1024
SKV = 1024
H_PER = 8
DH = 128
DM = 1024
BLK = 64
SCALE = 0.08838834764831843

PARTS = [(0, 384, 384), (384, 384, 768), (768, 256, 1024)]
RS_OFF = [
    [sum(PARTS[u][1] >> (r + 1) for u in range(t)) for t in range(3)]
    for r in range(3)
]
RS_TOT = [sum(PARTS[t][1] >> (r + 1) for t in range(3)) for r in range(3)]


def kernel(x, Wq, K_ext, V_ext, Wo):
    xb = x[0]
    k2 = K_ext[0].reshape(SKV, H_PER * DH)
    v2 = V_ext[0].reshape(SKV, H_PER * DH)

    def body(x_ref, k_ref, v_ref, wq_hbm, wo_hbm, out_ref,
             wq_vmem, wo_vmem, work, rs_recv0, rs_recv1, rs_recv2,
             w_sems, rs_send_sems, rs_recv_sems, ag_send_sems, ag_recv_sems):
        my_pos = lax.axis_index("i")

        wq_dma = pltpu.make_async_copy(
            wq_hbm.at[:, pl.ds(my_pos * (H_PER * DH), H_PER * DH)],
            wq_vmem, w_sems.at[0])
        wo_dma = pltpu.make_async_copy(
            wo_hbm.at[pl.ds(my_pos * (H_PER * DH), H_PER * DH), :],
            wo_vmem, w_sems.at[1])
        wq_dma.start()
        wo_dma.start()

        partners = [
            my_pos ^ 1,
            (my_pos & 4) | ((my_pos & 3) ^ 3),
            my_pos ^ 4,
        ]
        bits = [
            (my_pos ^ (my_pos >> 1)) & 1,
            (my_pos >> 1) & 1,
            (my_pos >> 2) & 1,
        ]
        rs_recv = [rs_recv0, rs_recv1, rs_recv2]

        barrier_sem = pltpu.get_barrier_semaphore()
        for p in partners:
            pl.semaphore_signal(
                barrier_sem, inc=1,
                device_id=(p,), device_id_type=pl.DeviceIdType.MESH,
            )
        pl.semaphore_wait(barrier_sem, 3)

        wq_dma.wait()
        wo_dma.wait()
        wq_bf = wq_vmem[:, :].astype(jnp.bfloat16)
        wo_bf = wo_vmem[:, :].astype(jnp.bfloat16)

        def compute_part(t):
            row0, nr, kvc = PARTS[t]
            xq = x_ref[row0:row0 + nr, :].astype(jnp.bfloat16)
            q = jnp.dot(
                xq, wq_bf, preferred_element_type=jnp.float32
            ).astype(jnp.bfloat16)
            qb = (row0 + lax.broadcasted_iota(jnp.int32, (nr, kvc), 0)) // BLK
            kb = lax.broadcasted_iota(jnp.int32, (nr, kvc), 1) // BLK
            mask = kb <= qb
            ctx_heads = []
            for h in range(H_PER):
                sl = slice(h * DH, (h + 1) * DH)
                k_h = k_ref[0:kvc, sl].astype(jnp.bfloat16)
                s = lax.dot_general(
                    q[:, sl], k_h, (((1,), (1,)), ((), ())),
                    preferred_element_type=jnp.float32,
                ) * SCALE
                s = jnp.where(mask, s, -1e9)
                m = jnp.max(s, axis=1, keepdims=True)
                e = jnp.exp(s - m)
                w = (e / jnp.sum(e, axis=1, keepdims=True)).astype(jnp.bfloat16)
                ctx_heads.append(jnp.dot(
                    w, v_ref[0:kvc, sl].astype(jnp.bfloat16),
                    preferred_element_type=jnp.float32,
                ).astype(jnp.bfloat16))
            ctx = jnp.concatenate(ctx_heads, axis=1)
            acc = jnp.dot(ctx, wo_bf, preferred_element_type=jnp.float32)
            work[row0:row0 + nr, :] = acc.astype(jnp.bfloat16)

        state = {t: (jnp.int32(PARTS[t][0]), PARTS[t][1]) for t in range(3)}
        rs_pending = {}
        ag_pending = {}

        def rs_send(t, r):
            d = (t + r) % 3
            start, sz = state[t]
            half = sz // 2
            b = bits[d]
            keep_start = start + b * half
            send_start = start + (1 - b) * half
            rdma = pltpu.make_async_remote_copy(
                src_ref=work.at[pl.ds(send_start, half), :],
                dst_ref=rs_recv[r].at[pl.ds(RS_OFF[r][t], half), :],
                send_sem=rs_send_sems.at[t, r],
                recv_sem=rs_recv_sems.at[t, r],
                device_id=(partners[d],),
                device_id_type=pl.DeviceIdType.MESH,
            )
            rdma.start()
            rs_pending[(t, r)] = (rdma, keep_start, half)
            state[t] = (keep_start, half)

        def rs_wait(t, r):
            rdma, keep_start, half = rs_pending[(t, r)]
            rdma.wait()
            work[pl.ds(keep_start, half), :] = (
                work[pl.ds(keep_start, half), :]
                + rs_recv[r][pl.ds(RS_OFF[r][t], half), :]
            )

        def ag_send(t, r):
            d = (t + 2 - r) % 3
            start, sz = state[t]
            rdma = pltpu.make_async_remote_copy(
                src_ref=work.at[pl.ds(start, sz), :],
                dst_ref=work.at[pl.ds(start, sz), :],
                send_sem=ag_send_sems.at[t, r],
                recv_sem=ag_recv_sems.at[t, r],
                device_id=(partners[d],),
                device_id_type=pl.DeviceIdType.MESH,
            )
            rdma.start()
            ag_pending[(t, r)] = rdma
            state[t] = (start - bits[d] * sz, 2 * sz)

        def ag_wait(t, r):
            ag_pending[(t, r)].wait()

        def emit_out(t):
            row0, nr, _ = PARTS[t]
            out_ref[0, row0:row0 + nr, :] = (
                work[row0:row0 + nr, :].astype(jnp.float32))

        compute_part(0); rs_send(0, 0)
        compute_part(1); rs_send(1, 0); rs_wait(0, 0); rs_send(0, 1)
        compute_part(2); rs_send(2, 0); rs_wait(1, 0); rs_send(1, 1)
        rs_wait(0, 1); rs_send(0, 2)
        rs_wait(2, 0); rs_send(2, 1); rs_wait(1, 1); rs_send(1, 2)
        rs_wait(0, 2); ag_send(0, 0)
        rs_wait(2, 1); rs_send(2, 2); rs_wait(1, 2); ag_send(1, 0)
        ag_wait(0, 0); ag_send(0, 1)
        rs_wait(2, 2); ag_send(2, 0); ag_wait(1, 0); ag_send(1, 1)
        ag_wait(0, 1); ag_send(0, 2)
        ag_wait(2, 0); ag_send(2, 1); ag_wait(1, 1); ag_send(1, 2)
        ag_wait(0, 2); emit_out(0)
        ag_wait(2, 1); ag_send(2, 2); ag_wait(1, 2); emit_out(1)
        ag_wait(2, 2); emit_out(2)

    out = pl.pallas_call(
        body,
        out_shape=jax.ShapeDtypeStruct((1, SQ, DM), jnp.float32),
        in_specs=[
            pl.BlockSpec(memory_space=pltpu.VMEM),
            pl.BlockSpec(memory_space=pltpu.VMEM),
            pl.BlockSpec(memory_space=pltpu.VMEM),
            pl.BlockSpec(memory_space=pl.ANY),
            pl.BlockSpec(memory_space=pl.ANY),
        ],
        out_specs=pl.BlockSpec(memory_space=pltpu.VMEM),
        scratch_shapes=[
            pltpu.VMEM((DM, H_PER * DH), jnp.float32),
            pltpu.VMEM((H_PER * DH, DM), jnp.float32),
            pltpu.VMEM((SQ, DM), jnp.bfloat16),
            pltpu.VMEM((RS_TOT[0], DM), jnp.bfloat16),
            pltpu.VMEM((RS_TOT[1], DM), jnp.bfloat16),
            pltpu.VMEM((RS_TOT[2], DM), jnp.bfloat16),
            pltpu.SemaphoreType.DMA((2,)),
            pltpu.SemaphoreType.DMA((3, 3)),
            pltpu.SemaphoreType.DMA((3, 3)),
            pltpu.SemaphoreType.DMA((3, 3)),
            pltpu.SemaphoreType.DMA((3, 3)),
        ],
        compiler_params=pltpu.CompilerParams(collective_id=0),
    )(xb, k2, v2, Wq, Wo)
    return out
